# baseline (speedup 1.0000x reference)
import functools

import jax
import jax.numpy as jnp
from jax import lax
from jax.experimental import pallas as pl
from jax.experimental.pallas import tpu as pltpu
from jax.experimental.pallas import tpu_sc as plsc

_S, _D, _H, _O = 2048, 1024, 1024, 1024
_E, _K = 8, 2
_TILE = 128
_CR = _S * _K + _E * _TILE
_NT = _CR // _TILE
_RTS = 256


def _router_body(x_ref, wr_ref, br_ref, logits_ref, idx_ref, wn_ref):
    xt = x_ref[...]
    l = jnp.dot(xt, wr_ref[...], preferred_element_type=jnp.float32) + br_ref[...]
    logits_ref[...] = l
    m = jnp.max(l, axis=1, keepdims=True)
    e = jnp.exp(l - m)
    w = e / jnp.sum(e, axis=1, keepdims=True)
    iota = lax.broadcasted_iota(jnp.int32, w.shape, 1)
    w1 = jnp.max(w, axis=1, keepdims=True)
    i1 = jnp.min(jnp.where(w == w1, iota, _E), axis=1, keepdims=True)
    wm = jnp.where(iota == i1, -1.0, w)
    w2 = jnp.max(wm, axis=1, keepdims=True)
    i2 = jnp.min(jnp.where(wm == w2, iota, _E), axis=1, keepdims=True)
    s = w1 + w2
    idx_ref[...] = jnp.concatenate([i1, i2], axis=1)
    wn_ref[...] = jnp.concatenate([w1 / s, w2 / s], axis=1)


def _router(x2, Wr, br):
    return pl.pallas_call(
        _router_body,
        grid=(_S // _RTS,),
        in_specs=[
            pl.BlockSpec((_RTS, _D), lambda i: (i, 0)),
            pl.BlockSpec((_D, _E), lambda i: (0, 0)),
            pl.BlockSpec((1, _E), lambda i: (0, 0)),
        ],
        out_specs=[
            pl.BlockSpec((_RTS, _E), lambda i: (i, 0)),
            pl.BlockSpec((_RTS, _K), lambda i: (i, 0)),
            pl.BlockSpec((_RTS, _K), lambda i: (i, 0)),
        ],
        out_shape=[
            jax.ShapeDtypeStruct((_S, _E), jnp.float32),
            jax.ShapeDtypeStruct((_S, _K), jnp.int32),
            jax.ShapeDtypeStruct((_S, _K), jnp.float32),
        ],
    )(x2, Wr, br.reshape(1, _E))


def _grouped_ffn_body(tile_eid_ref, xs_ref, gw_ref, vw_ref, ow_ref, ob_ref,
                      ws_ref, ys_ref):
    del tile_eid_ref
    xt = xs_ref[...]
    g = jnp.dot(xt, gw_ref[0], preferred_element_type=jnp.float32)
    v = jnp.dot(xt, vw_ref[0], preferred_element_type=jnp.float32)
    h = (g * jax.nn.sigmoid(g)) * v
    y = jnp.dot(h, ow_ref[0], preferred_element_type=jnp.float32) + ob_ref[0]
    ys_ref[...] = y * ws_ref[...]


def _grouped_ffn(tile_eid, xs, egW, evW, eoW, eob, w_row):
    grid_spec = pltpu.PrefetchScalarGridSpec(
        num_scalar_prefetch=1,
        grid=(_NT,),
        in_specs=[
            pl.BlockSpec((_TILE, _D), lambda i, te: (i, 0)),
            pl.BlockSpec((1, _D, _H), lambda i, te: (te[i], 0, 0)),
            pl.BlockSpec((1, _D, _H), lambda i, te: (te[i], 0, 0)),
            pl.BlockSpec((1, _H, _O), lambda i, te: (te[i], 0, 0)),
            pl.BlockSpec((1, 1, _O), lambda i, te: (te[i], 0, 0)),
            pl.BlockSpec((_TILE, 1), lambda i, te: (i, 0)),
        ],
        out_specs=pl.BlockSpec((_TILE, _O), lambda i, te: (i, 0)),
    )
    return pl.pallas_call(
        _grouped_ffn_body,
        grid_spec=grid_spec,
        out_shape=jax.ShapeDtypeStruct((_CR, _O), jnp.float32),
    )(tile_eid, xs, egW, evW, eoW, eob.reshape(_E, 1, _O), w_row)


def _shared_ffn_body(x_ref, gw_ref, vw_ref, ow_ref, ob_ref, y_ref):
    xt = x_ref[...]
    g = jnp.dot(xt, gw_ref[...], preferred_element_type=jnp.float32)
    v = jnp.dot(xt, vw_ref[...], preferred_element_type=jnp.float32)
    h = (g * jax.nn.sigmoid(g)) * v
    y_ref[...] = jnp.dot(h, ow_ref[...], preferred_element_type=jnp.float32) + ob_ref[...]


def _shared_ffn(x2, sgW, svW, soW, sob):
    return pl.pallas_call(
        _shared_ffn_body,
        grid=(_S // _RTS,),
        in_specs=[
            pl.BlockSpec((_RTS, _D), lambda i: (i, 0)),
            pl.BlockSpec((_D, _H), lambda i: (0, 0)),
            pl.BlockSpec((_D, _H), lambda i: (0, 0)),
            pl.BlockSpec((_H, _O), lambda i: (0, 0)),
            pl.BlockSpec((1, _O), lambda i: (0, 0)),
        ],
        out_specs=pl.BlockSpec((_RTS, _O), lambda i: (i, 0)),
        out_shape=jax.ShapeDtypeStruct((_S, _O), jnp.float32),
    )(x2, sgW, svW, soW, sob.reshape(1, _O))


_NC, _NS = 2, 16
_NW = _NC * _NS
_GB = _CR // _NW
_GCH = 40
_GNC = _GB // _GCH
_CB = _S // _NW
_CCH = 16
_CNC = _CB // _CCH

_SC_MESH = plsc.VectorSubcoreMesh(core_axis_name="c", subcore_axis_name="s")


def _dispatch_body(x_hbm, tok_hbm, xs_hbm, idx_v, rows_v, gsem):
    wid = lax.axis_index("s") * _NC + lax.axis_index("c")
    base = wid * _GB
    pltpu.sync_copy(tok_hbm.at[wid], idx_v)
    for c in range(_GNC):
        pltpu.async_copy(x_hbm.at[idx_v.at[c]], rows_v, gsem).wait()
        pltpu.sync_copy(rows_v, xs_hbm.at[pl.ds(base + c * _GCH, _GCH)])


_dispatch = functools.partial(
    pl.kernel,
    mesh=_SC_MESH,
    out_type=jax.ShapeDtypeStruct((_CR, _D), jnp.float32),
    scratch_types=[
        pltpu.VMEM((_GNC, _GCH), jnp.int32),
        pltpu.VMEM((_GCH, _D), jnp.float32),
        pltpu.SemaphoreType.DMA,
    ],
)(_dispatch_body)


def _combine_body(ys_hbm, ysh_hbm, p1_hbm, p2_hbm, out_hbm,
                  i1_v, i2_v, b1, b2, bsh, s1, s2, s3):
    wid = lax.axis_index("s") * _NC + lax.axis_index("c")
    base = wid * _CB
    pltpu.sync_copy(p1_hbm.at[wid], i1_v)
    pltpu.sync_copy(p2_hbm.at[wid], i2_v)
    for c in range(_CNC):
        off = base + c * _CCH
        d1 = pltpu.async_copy(ys_hbm.at[i1_v.at[c]], b1, s1)
        d2 = pltpu.async_copy(ys_hbm.at[i2_v.at[c]], b2, s2)
        d3 = pltpu.async_copy(ysh_hbm.at[pl.ds(off, _CCH)], bsh, s3)
        d1.wait()
        d2.wait()
        d3.wait()

        def _row(r, _):
            def _col(k, _):
                sl = pl.ds(k * 16, 16)
                bsh[r, sl] = b1[r, sl] + b2[r, sl] + bsh[r, sl]
                return 0
            return lax.fori_loop(0, _O // 16, _col, 0)

        lax.fori_loop(0, _CCH, _row, 0)
        pltpu.sync_copy(bsh, out_hbm.at[pl.ds(off, _CCH)])


_combine = functools.partial(
    pl.kernel,
    mesh=_SC_MESH,
    out_type=jax.ShapeDtypeStruct((_S, _O), jnp.float32),
    scratch_types=[
        pltpu.VMEM((_CNC, _CCH), jnp.int32),
        pltpu.VMEM((_CNC, _CCH), jnp.int32),
        pltpu.VMEM((_CCH, _O), jnp.float32),
        pltpu.VMEM((_CCH, _O), jnp.float32),
        pltpu.VMEM((_CCH, _O), jnp.float32),
        pltpu.SemaphoreType.DMA,
        pltpu.SemaphoreType.DMA,
        pltpu.SemaphoreType.DMA,
    ],
)(_combine_body)


def kernel(x, Wr, br, sgW, svW, soW, sob, egW, evW, eoW, eob):
    x2 = x.reshape(_S, _D)
    logits, topk_idx, wn = _router(x2, Wr, br)

    eid = topk_idx.reshape(-1)
    onehot = (eid[:, None] == jnp.arange(_E)[None, :]).astype(jnp.int32)
    counts = jnp.sum(onehot, axis=0)
    rank = jnp.take_along_axis(jnp.cumsum(onehot, axis=0) - onehot,
                               eid[:, None], axis=1)[:, 0]
    padded_counts = ((counts + _TILE - 1) // _TILE) * _TILE
    ends = jnp.cumsum(padded_counts)
    padded_offsets = ends - padded_counts
    pos = padded_offsets[eid] + rank
    row_token = jnp.zeros((_CR,), jnp.int32).at[pos].set(
        jnp.arange(_S * _K, dtype=jnp.int32) // _K)
    w_row = jnp.zeros((_CR, 1), jnp.float32).at[pos, 0].set(wn.reshape(-1))
    tile_eid = jnp.minimum(
        jnp.searchsorted(ends, jnp.arange(_NT, dtype=jnp.int32) * _TILE,
                         side="right"),
        _E - 1).astype(jnp.int32)

    xs = _dispatch(x2, row_token.reshape(_NW, _GNC, _GCH))

    ys = _grouped_ffn(tile_eid, xs, egW, evW, eoW, eob, w_row)
    ysh = _shared_ffn(x2, sgW, svW, soW, sob)

    p = pos.reshape(_S, _K)
    out = _combine(ys, ysh,
                   p[:, 0].reshape(_NW, _CNC, _CCH),
                   p[:, 1].reshape(_NW, _CNC, _CCH))

    return (out.reshape(1, _S, _O),
            logits.reshape(1, _S, _E),
            topk_idx.reshape(1, _S, _K))


# speedup vs baseline: 1.6927x; 1.6927x over previous
import functools

import jax
import jax.numpy as jnp
from jax import lax
from jax.experimental import pallas as pl
from jax.experimental.pallas import tpu as pltpu
from jax.experimental.pallas import tpu_sc as plsc

_S, _D, _H, _O = 2048, 1024, 1024, 1024
_E, _K = 8, 2
_TILE = 128
_CR = _S * _K + _E * _TILE
_NT = _CR // _TILE
_RTS = 256

_NC, _NS = 2, 16
_NW = _NC * _NS
_TB = _S // _NW


def _router_body(x_ref, wr_ref, br_ref,
                 logits_ref, idx_ref, wn1_ref, wn2_ref, pos_ref, te_ref):
    xt = x_ref[...]
    l = jnp.dot(xt, wr_ref[...], preferred_element_type=jnp.float32) + br_ref[...]
    logits_ref[...] = l
    m = jnp.max(l, axis=1, keepdims=True)
    e = jnp.exp(l - m)
    w = e / jnp.sum(e, axis=1, keepdims=True)
    iota8 = lax.broadcasted_iota(jnp.int32, (_S, _E), 1)
    w1 = jnp.max(w, axis=1, keepdims=True)
    i1 = jnp.min(jnp.where(w == w1, iota8, _E), axis=1, keepdims=True)
    wm = jnp.where(iota8 == i1, -1.0, w)
    w2 = jnp.max(wm, axis=1, keepdims=True)
    i2 = jnp.min(jnp.where(wm == w2, iota8, _E), axis=1, keepdims=True)
    s = w1 + w2
    idx_ref[...] = jnp.concatenate([i1, i2], axis=1)
    wn1_ref[...] = w1 / s
    wn2_ref[...] = w2 / s

    oh1 = (iota8 == i1).astype(jnp.float32)
    oh2 = (iota8 == i2).astype(jnp.float32)
    oh = oh1 + oh2
    rt = lax.broadcasted_iota(jnp.int32, (_S, _S), 0)
    ct = lax.broadcasted_iota(jnp.int32, (_S, _S), 1)
    tril = (ct < rt).astype(jnp.float32)
    pfx = jnp.dot(tril, oh, preferred_element_type=jnp.float32)
    rank1 = jnp.sum(pfx * oh1, axis=1, keepdims=True)
    rank2 = jnp.sum(pfx * oh2, axis=1, keepdims=True)
    counts = jnp.sum(oh, axis=0, keepdims=True)
    pcf = jnp.floor((counts + (_TILE - 1.0)) / _TILE) * _TILE
    pcb = jnp.broadcast_to(pcf, (_S, _E))
    po1 = jnp.sum(jnp.where(iota8 < i1, pcb, 0.0), axis=1, keepdims=True)
    po2 = jnp.sum(jnp.where(iota8 < i2, pcb, 0.0), axis=1, keepdims=True)
    pos1 = (po1 + rank1).astype(jnp.int32)
    pos2 = (po2 + rank2).astype(jnp.int32)
    pos_ref[...] = jnp.concatenate([pos1, pos2], axis=1)

    u8 = (lax.broadcasted_iota(jnp.int32, (_E, _E), 0)
          <= lax.broadcasted_iota(jnp.int32, (_E, _E), 1)).astype(jnp.float32)
    ends = jnp.dot(pcf, u8, preferred_element_type=jnp.float32)
    starts = jnp.broadcast_to(
        lax.broadcasted_iota(jnp.int32, (_NT, 1), 0).astype(jnp.float32)
        * _TILE, (_NT, _E))
    te = jnp.sum((jnp.broadcast_to(ends, (_NT, _E)) <= starts)
                 .astype(jnp.int32), axis=1, keepdims=True)
    te_ref[...] = jnp.minimum(te, _E - 1)


def _router(x2, Wr, br):
    return pl.pallas_call(
        _router_body,
        out_shape=[
            jax.ShapeDtypeStruct((_S, _E), jnp.float32),
            jax.ShapeDtypeStruct((_S, _K), jnp.int32),
            jax.ShapeDtypeStruct((_S, 1), jnp.float32),
            jax.ShapeDtypeStruct((_S, 1), jnp.float32),
            jax.ShapeDtypeStruct((_S, _K), jnp.int32),
            jax.ShapeDtypeStruct((_NT, 1), jnp.int32),
        ],
    )(x2, Wr, br.reshape(1, _E))


def _dispatch_body(x_hbm, pos_hbm, xs_hbm, idx_v, xbuf, s1, s2):
    wid = lax.axis_index("s") * _NC + lax.axis_index("c")
    base = wid * _TB
    pltpu.sync_copy(pos_hbm.at[wid], idx_v)
    pltpu.sync_copy(x_hbm.at[pl.ds(base, _TB)], xbuf)
    d1 = pltpu.async_copy(xbuf, xs_hbm.at[idx_v.at[0]], s1)
    d2 = pltpu.async_copy(xbuf, xs_hbm.at[idx_v.at[1]], s2)
    d1.wait()
    d2.wait()


def _dispatch(x2, pos_w):
    return pl.kernel(
        _dispatch_body,
        mesh=plsc.VectorSubcoreMesh(core_axis_name="c", subcore_axis_name="s"),
        out_type=jax.ShapeDtypeStruct((_CR, _D), jnp.float32),
        scratch_types=[
            pltpu.VMEM((_K, _TB), jnp.int32),
            pltpu.VMEM((_TB, _D), jnp.float32),
            pltpu.SemaphoreType.DMA,
            pltpu.SemaphoreType.DMA,
        ],
    )(x2, pos_w)


def _gather2_body(ys_hbm, pos_hbm, g1_hbm, g2_hbm, idx_v, buf, s1):
    wid = lax.axis_index("s") * _NC + lax.axis_index("c")
    base = wid * _TB
    pltpu.sync_copy(pos_hbm.at[wid], idx_v)
    half = _TB // 2
    for k, out_hbm in ((0, g1_hbm), (1, g2_hbm)):
        for c in range(2):
            pltpu.async_copy(
                ys_hbm.at[idx_v.at[k, pl.ds(c * half, half)]], buf, s1).wait()
            pltpu.sync_copy(buf, out_hbm.at[pl.ds(base + c * half, half)])


def _gather2(ys, pos_w):
    return pl.kernel(
        _gather2_body,
        mesh=plsc.VectorSubcoreMesh(core_axis_name="c", subcore_axis_name="s"),
        out_type=[
            jax.ShapeDtypeStruct((_S, _O), jnp.float32),
            jax.ShapeDtypeStruct((_S, _O), jnp.float32),
        ],
        scratch_types=[
            pltpu.VMEM((_K, _TB), jnp.int32),
            pltpu.VMEM((_TB // 2, _O), jnp.float32),
            pltpu.SemaphoreType.DMA,
        ],
    )(ys, pos_w)


def _grouped_ffn_body(te_ref, xs_ref, gw_ref, vw_ref, ow_ref, ob_ref, ys_ref):
    del te_ref
    xt = xs_ref[...]
    g = jnp.dot(xt, gw_ref[0], preferred_element_type=jnp.float32)
    v = jnp.dot(xt, vw_ref[0], preferred_element_type=jnp.float32)
    h = (g * jax.nn.sigmoid(g)) * v
    ys_ref[...] = jnp.dot(h, ow_ref[0], preferred_element_type=jnp.float32) + ob_ref[0]


def _grouped_ffn(te, xs, egW, evW, eoW, eob):
    grid_spec = pltpu.PrefetchScalarGridSpec(
        num_scalar_prefetch=1,
        grid=(_NT,),
        in_specs=[
            pl.BlockSpec((_TILE, _D), lambda i, te: (i, 0)),
            pl.BlockSpec((1, _D, _H), lambda i, te: (te[i], 0, 0)),
            pl.BlockSpec((1, _D, _H), lambda i, te: (te[i], 0, 0)),
            pl.BlockSpec((1, _H, _O), lambda i, te: (te[i], 0, 0)),
            pl.BlockSpec((1, 1, _O), lambda i, te: (te[i], 0, 0)),
        ],
        out_specs=pl.BlockSpec((_TILE, _O), lambda i, te: (i, 0)),
    )
    return pl.pallas_call(
        _grouped_ffn_body,
        grid_spec=grid_spec,
        out_shape=jax.ShapeDtypeStruct((_CR, _O), jnp.float32),
    )(te, xs, egW, evW, eoW, eob.reshape(_E, 1, _O))


def _shared_combine_body(x_ref, gw_ref, vw_ref, ow_ref, ob_ref,
                         g1_ref, g2_ref, wn1_ref, wn2_ref, y_ref):
    xt = x_ref[...]
    g = jnp.dot(xt, gw_ref[...], preferred_element_type=jnp.float32)
    v = jnp.dot(xt, vw_ref[...], preferred_element_type=jnp.float32)
    h = (g * jax.nn.sigmoid(g)) * v
    y = jnp.dot(h, ow_ref[...], preferred_element_type=jnp.float32) + ob_ref[...]
    y_ref[...] = y + wn1_ref[...] * g1_ref[...] + wn2_ref[...] * g2_ref[...]


def _shared_combine(x2, sgW, svW, soW, sob, g1, g2, wn1, wn2):
    return pl.pallas_call(
        _shared_combine_body,
        grid=(_S // _RTS,),
        in_specs=[
            pl.BlockSpec((_RTS, _D), lambda i: (i, 0)),
            pl.BlockSpec((_D, _H), lambda i: (0, 0)),
            pl.BlockSpec((_D, _H), lambda i: (0, 0)),
            pl.BlockSpec((_H, _O), lambda i: (0, 0)),
            pl.BlockSpec((1, _O), lambda i: (0, 0)),
            pl.BlockSpec((_RTS, _O), lambda i: (i, 0)),
            pl.BlockSpec((_RTS, _O), lambda i: (i, 0)),
            pl.BlockSpec((_RTS, 1), lambda i: (i, 0)),
            pl.BlockSpec((_RTS, 1), lambda i: (i, 0)),
        ],
        out_specs=pl.BlockSpec((_RTS, _O), lambda i: (i, 0)),
        out_shape=jax.ShapeDtypeStruct((_S, _O), jnp.float32),
    )(x2, sgW, svW, soW, sob.reshape(1, _O), g1, g2, wn1, wn2)


def kernel(x, Wr, br, sgW, svW, soW, sob, egW, evW, eoW, eob):
    x2 = x.reshape(_S, _D)
    logits, topk_idx, wn1, wn2, pos, te = _router(x2, Wr, br)

    pos_w = pos.reshape(_NW, _TB, _K).transpose(0, 2, 1)

    xs = _dispatch(x2, pos_w)
    ys = _grouped_ffn(te.reshape(_NT), xs, egW, evW, eoW, eob)
    g1, g2 = _gather2(ys, pos_w)
    out = _shared_combine(x2, sgW, svW, soW, sob, g1, g2, wn1, wn2)

    return (out.reshape(1, _S, _O),
            logits.reshape(1, _S, _E),
            topk_idx.reshape(1, _S, _K))
